# baseline (speedup 1.0000x reference)
import functools

import jax
import jax.numpy as jnp
from jax import lax
from jax.experimental import pallas as pl
from jax.experimental.pallas import tpu as pltpu
from jax.experimental.pallas import tpu_sc as plsc

_info = plsc.get_sparse_core_info()
_NC, _NS = _info.num_cores, _info.num_subcores
_NW = _NC * _NS

_B = 16384
_S = 200
_D = 64
_TS = _S // 8
_BC = _B // 128
_UNITS = _TS * _BC
_UPW = _UNITS // _NW


def _transpose_half(rows_ref, tr_v):
    lane = lax.iota(jnp.int32, 16)

    for rr in range(4):
        def jbody(j, carry, rr=rr):
            cidx = jnp.full((16,), j, jnp.int32)
            for k in range(8):
                ridx = lane + (rr * 128 + k * 16)
                v = plsc.load_gather(rows_ref, [ridx, cidx])
                tr_v[rr, j, pl.ds(k * 16, 16)] = v
            return carry

        lax.fori_loop(0, 64, jbody, 0)


def _make_lookup():
    mesh = plsc.VectorSubcoreMesh(core_axis_name="c", subcore_axis_name="s")

    @functools.partial(
        pl.kernel,
        mesh=mesh,
        out_type=jax.ShapeDtypeStruct((_S, 8, _BC, 8, 128), jnp.float32),
        scratch_types=[
            pltpu.VMEM((2, 1024), jnp.int32),
            pltpu.VMEM((2, 512, _D), jnp.float32),
            pltpu.VMEM((4, _D, 128), jnp.float32),
            pltpu.SemaphoreType.DMA((2,)),
            pltpu.SemaphoreType.DMA((2,)),
            pltpu.SemaphoreType.DMA,
        ],
        compiler_params=pltpu.CompilerParams(use_tc_tiling_on_sc=False,
                                             needs_layout_passes=False),
    )
    def lookup(table_hbm, x4_hbm, o5_hbm, idx_v, rows_v, tr_v, s_idx, s_gat, s_out):
        wid = lax.axis_index("s") * _NC + lax.axis_index("c")
        g0 = wid * _UPW

        def unit_tc(u):
            g = g0 + u
            return g // _BC, g % _BC

        def idx_load(u, pu):
            t, c = unit_tc(u)
            pltpu.async_copy(x4_hbm.at[t, c], idx_v.at[pu], s_idx.at[pu])

        def idx_wait(pu):
            pltpu.make_async_copy(x4_hbm.at[0, 0], idx_v.at[pu],
                                  s_idx.at[pu]).wait()

        def gather(h, pu):
            pltpu.async_copy(
                table_hbm.at[idx_v.at[pu, pl.ds(512 * h, 512)]],
                rows_v.at[h], s_gat.at[h])

        def gather_wait(h):
            pltpu.make_async_copy(
                table_hbm.at[idx_v.at[0, pl.ds(0, 512)]], rows_v.at[h],
                s_gat.at[h]).wait()

        def writes(u, h):
            t, c = unit_tc(u)
            for rr in range(4):
                s = 8 * t + 4 * h + rr
                for jo in range(8):
                    pltpu.async_copy(tr_v.at[rr, pl.ds(jo * 8, 8)],
                                     o5_hbm.at[s, jo, c], s_out)

        def writes_drain():
            for _ in range(32):
                pltpu.make_async_copy(tr_v.at[0, pl.ds(0, 8)],
                                      o5_hbm.at[0, 0, 0], s_out).wait()

        idx_load(0, 0)
        idx_wait(0)
        gather(0, 0)
        idx_load(1, 1)

        def half_step(u, pu, h, first):
            gather_wait(h)
            if h == 0:
                gather(1, pu)
            else:
                @pl.when(u < _UPW - 1)
                def _():
                    idx_wait(1 - pu)
                    gather(0, 1 - pu)

                @pl.when(u < _UPW - 2)
                def _():
                    idx_load(u + 2, pu)

            if first:
                @pl.when(u > 0)
                def _():
                    writes_drain()
            else:
                writes_drain()
            _transpose_half(rows_v.at[h], tr_v)
            writes(u, h)

        def mega(m, carry):
            u0 = 2 * m
            half_step(u0, 0, 0, True)
            half_step(u0, 0, 1, False)
            half_step(u0 + 1, 1, 0, False)
            half_step(u0 + 1, 1, 1, False)
            return carry

        lax.fori_loop(0, _UPW // 2, mega, 0)
        writes_drain()

    return lookup


def kernel(x, table):
    x4 = jnp.transpose(x.reshape(128, 128, _TS, 8), (2, 0, 3, 1))
    x4 = x4.reshape(_TS, 128, 1024).astype(jnp.int32)
    o5 = _make_lookup()(table, x4)
    out = jnp.transpose(o5, (2, 4, 0, 1, 3))
    return out.reshape(_B, _S, _D)


# speedup vs baseline: 1.5773x; 1.5773x over previous
import functools

import jax
import jax.numpy as jnp
from jax import lax
from jax.experimental import pallas as pl
from jax.experimental.pallas import tpu as pltpu
from jax.experimental.pallas import tpu_sc as plsc

_info = plsc.get_sparse_core_info()
_NC, _NS = _info.num_cores, _info.num_subcores
_NW = _NC * _NS

_B = 16384
_S = 200
_D = 64
_TS = _S // 8
_BC = _B // 128
_UNITS = _TS * _BC
_UPW = _UNITS // _NW


def _transpose_half(rows_ref, tr_v):
    lane = lax.iota(jnp.int32, 16)

    for rr in range(4):
        @plsc.parallel_loop(0, 64, step=1, unroll=8)
        def _(j, rr=rr):
            cidx = jnp.full((16,), j, jnp.int32)
            for k in range(8):
                ridx = lane + (rr * 128 + k * 16)
                v = plsc.load_gather(rows_ref, [ridx, cidx])
                tr_v[rr, j, pl.ds(k * 16, 16)] = v


def _make_lookup():
    mesh = plsc.VectorSubcoreMesh(core_axis_name="c", subcore_axis_name="s")

    @functools.partial(
        pl.kernel,
        mesh=mesh,
        out_type=jax.ShapeDtypeStruct((_S, 8, _BC, 8, 128), jnp.float32),
        scratch_types=[
            pltpu.VMEM((2, 1024), jnp.int32),
            pltpu.VMEM((2, 512, _D), jnp.float32),
            pltpu.VMEM((4, _D, 128), jnp.float32),
            pltpu.SemaphoreType.DMA((2,)),
            pltpu.SemaphoreType.DMA((2,)),
            pltpu.SemaphoreType.DMA,
        ],
        compiler_params=pltpu.CompilerParams(use_tc_tiling_on_sc=False,
                                             needs_layout_passes=False),
    )
    def lookup(table_hbm, x4_hbm, o5_hbm, idx_v, rows_v, tr_v, s_idx, s_gat, s_out):
        wid = lax.axis_index("s") * _NC + lax.axis_index("c")
        g0 = wid * _UPW

        def unit_tc(u):
            g = g0 + u
            return g // _BC, g % _BC

        def idx_load(u, pu):
            t, c = unit_tc(u)
            pltpu.async_copy(x4_hbm.at[t, c], idx_v.at[pu], s_idx.at[pu])

        def idx_wait(pu):
            pltpu.make_async_copy(x4_hbm.at[0, 0], idx_v.at[pu],
                                  s_idx.at[pu]).wait()

        def gather(h, pu):
            pltpu.async_copy(
                table_hbm.at[idx_v.at[pu, pl.ds(512 * h, 512)]],
                rows_v.at[h], s_gat.at[h])

        def gather_wait(h):
            pltpu.make_async_copy(
                table_hbm.at[idx_v.at[0, pl.ds(0, 512)]], rows_v.at[h],
                s_gat.at[h]).wait()

        def writes(u, h):
            t, c = unit_tc(u)
            for rr in range(4):
                s = 8 * t + 4 * h + rr
                for jo in range(8):
                    pltpu.async_copy(tr_v.at[rr, pl.ds(jo * 8, 8)],
                                     o5_hbm.at[s, jo, c], s_out)

        def writes_drain():
            for _ in range(32):
                pltpu.make_async_copy(tr_v.at[0, pl.ds(0, 8)],
                                      o5_hbm.at[0, 0, 0], s_out).wait()

        idx_load(0, 0)
        idx_wait(0)
        gather(0, 0)
        idx_load(1, 1)

        def half_step(u, pu, h, first):
            gather_wait(h)
            if h == 0:
                gather(1, pu)
            else:
                @pl.when(u < _UPW - 1)
                def _():
                    idx_wait(1 - pu)
                    gather(0, 1 - pu)

                @pl.when(u < _UPW - 2)
                def _():
                    idx_load(u + 2, pu)

            if first:
                @pl.when(u > 0)
                def _():
                    writes_drain()
            else:
                writes_drain()
            _transpose_half(rows_v.at[h], tr_v)
            writes(u, h)

        def mega(m, carry):
            u0 = 2 * m
            half_step(u0, 0, 0, True)
            half_step(u0, 0, 1, False)
            half_step(u0 + 1, 1, 0, False)
            half_step(u0 + 1, 1, 1, False)
            return carry

        lax.fori_loop(0, _UPW // 2, mega, 0)
        writes_drain()

    return lookup


def kernel(x, table):
    x4 = jnp.transpose(x.reshape(128, 128, _TS, 8), (2, 0, 3, 1))
    x4 = x4.reshape(_TS, 128, 1024).astype(jnp.int32)
    o5 = _make_lookup()(table, x4)
    out = jnp.transpose(o5, (2, 4, 0, 1, 3))
    return out.reshape(_B, _S, _D)
